# baseline (speedup 1.0000x reference)
import functools

import jax
import jax.numpy as jnp
from jax import lax
from jax.experimental import pallas as pl
from jax.experimental.pallas import tpu as pltpu
from jax.experimental.pallas import tpu_sc as plsc

B = 16384
D = 32
NC = 2
NS = 16
NW = NC * NS
BPW = B // NW
CHUNK = 128
NCH = BPW // CHUNK
LANES = 16


def _cmf_body(users_hbm, items_hbm, uemb_hbm, iemb_hbm, out_hbm,
              uidx_v, iidx_v, urows_v, irows_v, out_v, usem, isem):
    wid = lax.axis_index("s") * NC + lax.axis_index("c")

    pltpu.sync_copy(users_hbm.at[pl.ds(wid * NCH, NCH)], uidx_v)
    pltpu.sync_copy(items_hbm.at[pl.ds(wid * NCH, NCH)], iidx_v)

    ucopies = [
        pltpu.async_copy(uemb_hbm.at[uidx_v.at[j]],
                         urows_v.at[pl.ds(j * CHUNK, CHUNK)], usem)
        for j in range(NCH)
    ]
    icopies = [
        pltpu.async_copy(iemb_hbm.at[iidx_v.at[j]],
                         irows_v.at[pl.ds(j * CHUNK, CHUNK)], isem)
        for j in range(NCH)
    ]
    for c in ucopies:
        c.wait()
    for c in icopies:
        c.wait()

    lane = lax.iota(jnp.int32, LANES)

    def chunk16(c, carry):
        base = c * LANES
        acc = jnp.zeros((LANES,), jnp.float32)
        for l in range(LANES):
            b = base + l
            u0 = urows_v[b, pl.ds(0, LANES)]
            u1 = urows_v[b, pl.ds(LANES, LANES)]
            v0 = irows_v[b, pl.ds(0, LANES)]
            v1 = irows_v[b, pl.ds(LANES, LANES)]
            s = u0 * v0 + u1 * v1
            acc = jnp.where(lane == l, jnp.sum(s), acc)
        out_v[pl.ds(base, LANES)] = acc
        return carry

    lax.fori_loop(0, BPW // LANES, chunk16, 0)

    pltpu.sync_copy(out_v, out_hbm.at[pl.ds(wid * BPW, BPW)])


@jax.jit
def kernel(users, items, user_emb, item_emb):
    users2 = users.astype(jnp.int32).reshape(B // CHUNK, CHUNK)
    items2 = items.astype(jnp.int32).reshape(B // CHUNK, CHUNK)
    mesh = plsc.VectorSubcoreMesh(core_axis_name="c", subcore_axis_name="s")
    run = pl.kernel(
        _cmf_body,
        out_type=jax.ShapeDtypeStruct((B,), jnp.float32),
        mesh=mesh,
        compiler_params=pltpu.CompilerParams(
            needs_layout_passes=False, use_tc_tiling_on_sc=False),
        scratch_types=[
            pltpu.VMEM((NCH, CHUNK), jnp.int32),
            pltpu.VMEM((NCH, CHUNK), jnp.int32),
            pltpu.VMEM((BPW, D), jnp.float32),
            pltpu.VMEM((BPW, D), jnp.float32),
            pltpu.VMEM((BPW,), jnp.float32),
            pltpu.SemaphoreType.DMA,
            pltpu.SemaphoreType.DMA,
        ],
    )
    return run(users2, items2, user_emb, item_emb)
